# Initial kernel scaffold; baseline (speedup 1.0000x reference)
import jax
import jax.numpy as jnp
from jax.experimental import pallas as pl


def kernel(x, W1, b1, W2, b2, keys):
    raise NotImplementedError("write your pallas kernel here")


# speedup vs baseline: 1.5523x; 1.5523x over previous
import functools

import jax
import jax.numpy as jnp
from jax.experimental import pallas as pl
from jax.experimental.pallas import tpu as pltpu

_N = 8192
_E = 16
_K = 2
_BLW = 0.01
_BLK = 512


def _router_kernel(x_ref, w1t_ref, b1_ref, w2t_ref, b2_ref, keyst_ref,
                   keys_ref, idx_ref, scr_ref, sel_ref, load_ref, bal_ref,
                   spe_acc):
    i = pl.program_id(0)
    nblocks = pl.num_programs(0)
    blk = x_ref.shape[0]
    e = keys_ref.shape[0]
    d_out = keys_ref.shape[1]

    hp = jax.lax.Precision.DEFAULT
    h1 = jnp.maximum(
        jnp.dot(x_ref[...], w1t_ref[...], precision=hp) + b1_ref[...], 0.0)
    h = jnp.dot(h1, w2t_ref[...], precision=hp) + b2_ref[...]
    logits = jnp.dot(h, keyst_ref[...], precision=hp)

    m = jnp.max(logits, axis=-1, keepdims=True)
    ex = jnp.exp(logits - m)
    denom = jnp.sum(ex, axis=-1, keepdims=True)
    p = ex / denom

    cols = jax.lax.broadcasted_iota(jnp.int32, (blk, e), 1)
    i1 = jnp.min(jnp.where(logits == m, cols, e), axis=-1, keepdims=True)
    masked = jnp.where(cols == i1, -jnp.inf, logits)
    m2 = jnp.max(masked, axis=-1, keepdims=True)
    i2 = jnp.min(jnp.where(masked == m2, cols, e), axis=-1, keepdims=True)

    oh1 = (cols == i1).astype(jnp.float32)
    oh2 = (cols == i2).astype(jnp.float32)
    p1 = jnp.sum(p * oh1, axis=-1, keepdims=True)
    p2 = jnp.sum(p * oh2, axis=-1, keepdims=True)
    tot = p1 + p2

    idx_ref[...] = jnp.concatenate([i1, i2], axis=1)
    scr_ref[...] = jnp.concatenate([p1 / tot, p2 / tot], axis=1)

    sel_ref[:, :d_out] = jnp.dot(oh1, keys_ref[...], precision=hp)
    sel_ref[:, d_out:] = jnp.dot(oh2, keys_ref[...], precision=hp)

    tpe_blk = jnp.sum(oh1 + oh2, axis=0, keepdims=True)
    spe_blk = jnp.sum(p, axis=0, keepdims=True)

    @pl.when(i == 0)
    def _init():
        load_ref[...] = tpe_blk
        spe_acc[...] = spe_blk

    @pl.when(i != 0)
    def _acc():
        load_ref[...] += tpe_blk
        spe_acc[...] += spe_blk

    @pl.when(i == nblocks - 1)
    def _fin():
        n = jnp.float32(_N)
        tpe = load_ref[...] / n
        spe = spe_acc[...] / n
        bal_ref[...] = jnp.sum(tpe * spe).reshape(1, 1) * (_BLW * _E)


@functools.partial(jax.jit, static_argnames=("interpret",))
def kernel(x, W1, b1, W2, b2, keys, interpret=False):
    n, d_in = x.shape
    d_hid = W1.shape[0]
    d_out = W2.shape[0]
    e = keys.shape[0]
    blk = _BLK
    grid = n // blk

    w1t = W1.T
    w2t = W2.T
    keyst = keys.T
    b1r = b1[None, :]
    b2r = b2[None, :]

    out_shapes = (
        jax.ShapeDtypeStruct((n, _K), jnp.int32),
        jax.ShapeDtypeStruct((n, _K), jnp.float32),
        jax.ShapeDtypeStruct((n, _K * d_out), jnp.float32),
        jax.ShapeDtypeStruct((1, e), jnp.float32),
        jax.ShapeDtypeStruct((1, 1), jnp.float32),
    )
    in_specs = [
        pl.BlockSpec((blk, d_in), lambda i: (i, 0)),
        pl.BlockSpec((d_in, d_hid), lambda i: (0, 0)),
        pl.BlockSpec((1, d_hid), lambda i: (0, 0)),
        pl.BlockSpec((d_hid, d_out), lambda i: (0, 0)),
        pl.BlockSpec((1, d_out), lambda i: (0, 0)),
        pl.BlockSpec((d_out, e), lambda i: (0, 0)),
        pl.BlockSpec((e, d_out), lambda i: (0, 0)),
    ]
    out_specs = (
        pl.BlockSpec((blk, _K), lambda i: (i, 0)),
        pl.BlockSpec((blk, _K), lambda i: (i, 0)),
        pl.BlockSpec((blk, _K * d_out), lambda i: (i, 0)),
        pl.BlockSpec((1, e), lambda i: (0, 0)),
        pl.BlockSpec((1, 1), lambda i: (0, 0)),
    )

    idx, scr, sel, load2d, bal = pl.pallas_call(
        _router_kernel,
        grid=(grid,),
        in_specs=in_specs,
        out_specs=out_specs,
        out_shape=out_shapes,
        scratch_shapes=[pltpu.VMEM((1, e), jnp.float32)],
        interpret=interpret,
    )(x, w1t, b1r, w2t, b2r, keyst, keys)

    top_k_indices = idx
    top_k_scores = scr
    selected_keys = sel.reshape(n, _K, d_out)
    load = load2d.reshape(e)
    balance_loss = bal.reshape(())
    importance = jnp.float32(0.0)
    return (top_k_indices, top_k_scores, balance_loss, load, importance,
            selected_keys)


# speedup vs baseline: 2.7819x; 1.7921x over previous
import functools

import jax
import jax.numpy as jnp
from jax.experimental import pallas as pl
from jax.experimental.pallas import tpu as pltpu

_N = 8192
_E = 16
_K = 2
_BLW = 0.01
_BLK = 512


def _router_kernel(x_ref, w1t_ref, b1_ref, w2t_ref, b2_ref, keyst_ref,
                   keys_ref, idx_ref, scr_ref, sel_ref, load_ref, bal_ref,
                   spe_acc):
    i = pl.program_id(0)
    nblocks = pl.num_programs(0)
    blk = x_ref.shape[0]
    e = keys_ref.shape[0]
    d_out = keys_ref.shape[1]

    hp = jax.lax.Precision.DEFAULT
    h1 = jnp.maximum(
        jnp.dot(x_ref[...], w1t_ref[...], precision=hp) + b1_ref[...], 0.0)
    h = jnp.dot(h1, w2t_ref[...], precision=hp) + b2_ref[...]
    logits = jnp.dot(h, keyst_ref[...], precision=hp)

    m = jnp.max(logits, axis=-1, keepdims=True)
    ex = jnp.exp(logits - m)
    denom = jnp.sum(ex, axis=-1, keepdims=True)
    p = ex / denom

    cols = jax.lax.broadcasted_iota(jnp.int32, (blk, e), 1)
    i1 = jnp.min(jnp.where(logits == m, cols, e), axis=-1, keepdims=True)
    masked = jnp.where(cols == i1, -jnp.inf, logits)
    m2 = jnp.max(masked, axis=-1, keepdims=True)
    i2 = jnp.min(jnp.where(masked == m2, cols, e), axis=-1, keepdims=True)

    oh1 = (cols == i1).astype(jnp.float32)
    oh2 = (cols == i2).astype(jnp.float32)
    p1 = jnp.sum(p * oh1, axis=-1, keepdims=True)
    p2 = jnp.sum(p * oh2, axis=-1, keepdims=True)
    tot = p1 + p2

    idx_ref[...] = jnp.concatenate([i1, i2], axis=1)
    scr_ref[...] = jnp.concatenate([p1 / tot, p2 / tot], axis=1)

    sel_ref[:, 0, :] = jnp.dot(oh1, keys_ref[...], precision=hp)
    sel_ref[:, 1, :] = jnp.dot(oh2, keys_ref[...], precision=hp)

    tpe_blk = jnp.sum(oh1 + oh2, axis=0, keepdims=True)
    spe_blk = jnp.sum(p, axis=0, keepdims=True)

    @pl.when(i == 0)
    def _init():
        load_ref[...] = tpe_blk
        spe_acc[...] = spe_blk

    @pl.when(i != 0)
    def _acc():
        load_ref[...] += tpe_blk
        spe_acc[...] += spe_blk

    @pl.when(i == nblocks - 1)
    def _fin():
        n = jnp.float32(_N)
        tpe = load_ref[...] / n
        spe = spe_acc[...] / n
        bal_ref[...] = jnp.sum(tpe * spe).reshape(1, 1) * (_BLW * _E)


@functools.partial(jax.jit, static_argnames=("interpret",))
def kernel(x, W1, b1, W2, b2, keys, interpret=False):
    n, d_in = x.shape
    d_hid = W1.shape[0]
    d_out = W2.shape[0]
    e = keys.shape[0]
    blk = _BLK
    grid = n // blk

    w1t = W1.T
    w2t = W2.T
    keyst = keys.T
    b1r = b1[None, :]
    b2r = b2[None, :]

    out_shapes = (
        jax.ShapeDtypeStruct((n, _K), jnp.int32),
        jax.ShapeDtypeStruct((n, _K), jnp.float32),
        jax.ShapeDtypeStruct((n, _K, d_out), jnp.float32),
        jax.ShapeDtypeStruct((1, e), jnp.float32),
        jax.ShapeDtypeStruct((1, 1), jnp.float32),
    )
    in_specs = [
        pl.BlockSpec((blk, d_in), lambda i: (i, 0)),
        pl.BlockSpec((d_in, d_hid), lambda i: (0, 0)),
        pl.BlockSpec((1, d_hid), lambda i: (0, 0)),
        pl.BlockSpec((d_hid, d_out), lambda i: (0, 0)),
        pl.BlockSpec((1, d_out), lambda i: (0, 0)),
        pl.BlockSpec((d_out, e), lambda i: (0, 0)),
        pl.BlockSpec((e, d_out), lambda i: (0, 0)),
    ]
    out_specs = (
        pl.BlockSpec((blk, _K), lambda i: (i, 0)),
        pl.BlockSpec((blk, _K), lambda i: (i, 0)),
        pl.BlockSpec((blk, _K, d_out), lambda i: (i, 0, 0)),
        pl.BlockSpec((1, e), lambda i: (0, 0)),
        pl.BlockSpec((1, 1), lambda i: (0, 0)),
    )

    idx, scr, sel, load2d, bal = pl.pallas_call(
        _router_kernel,
        grid=(grid,),
        in_specs=in_specs,
        out_specs=out_specs,
        out_shape=out_shapes,
        scratch_shapes=[pltpu.VMEM((1, e), jnp.float32)],
        interpret=interpret,
    )(x, w1t, b1r, w2t, b2r, keyst, keys)

    top_k_indices = idx
    top_k_scores = scr
    selected_keys = sel
    load = load2d.reshape(e)
    balance_loss = bal.reshape(())
    importance = jnp.float32(0.0)
    return (top_k_indices, top_k_scores, balance_loss, load, importance,
            selected_keys)


# speedup vs baseline: 3.0815x; 1.1077x over previous
import functools

import jax
import jax.numpy as jnp
from jax.experimental import pallas as pl
from jax.experimental.pallas import tpu as pltpu

_N = 8192
_E = 16
_K = 2
_BLW = 0.01
_BLK = 512


def _dot_t(a, b):
    return jax.lax.dot_general(
        a, b, (((1,), (1,)), ((), ())),
        precision=jax.lax.Precision.DEFAULT,
        preferred_element_type=jnp.float32)


def _router_kernel(x_ref, w1_ref, b1_ref, w2_ref, b2_ref,
                   keys_ref, idx_ref, scr_ref, sel_ref, load_ref, bal_ref,
                   spe_acc):
    i = pl.program_id(0)
    nblocks = pl.num_programs(0)
    blk = x_ref.shape[0]
    e = keys_ref.shape[0]
    d_out = keys_ref.shape[1]

    hp = jax.lax.Precision.DEFAULT
    h1 = jnp.maximum(_dot_t(x_ref[...], w1_ref[...]) + b1_ref[...], 0.0)
    h = _dot_t(h1, w2_ref[...]) + b2_ref[...]
    logits = _dot_t(h, keys_ref[...])

    m = jnp.max(logits, axis=-1, keepdims=True)
    ex = jnp.exp(logits - m)
    denom = jnp.sum(ex, axis=-1, keepdims=True)
    p = ex / denom

    cols = jax.lax.broadcasted_iota(jnp.int32, (blk, e), 1)
    i1 = jnp.min(jnp.where(logits == m, cols, e), axis=-1, keepdims=True)
    masked = jnp.where(cols == i1, -jnp.inf, logits)
    m2 = jnp.max(masked, axis=-1, keepdims=True)
    i2 = jnp.min(jnp.where(masked == m2, cols, e), axis=-1, keepdims=True)

    oh1 = (cols == i1).astype(jnp.float32)
    oh2 = (cols == i2).astype(jnp.float32)
    p1 = jnp.sum(p * oh1, axis=-1, keepdims=True)
    p2 = jnp.sum(p * oh2, axis=-1, keepdims=True)
    tot = p1 + p2

    idx_ref[...] = jnp.concatenate([i1, i2], axis=1)
    scr_ref[...] = jnp.concatenate([p1 / tot, p2 / tot], axis=1)

    sel_ref[:, 0, :] = jnp.dot(oh1, keys_ref[...], precision=hp)
    sel_ref[:, 1, :] = jnp.dot(oh2, keys_ref[...], precision=hp)

    tpe_blk = jnp.sum(oh1 + oh2, axis=0, keepdims=True)
    spe_blk = jnp.sum(p, axis=0, keepdims=True)

    @pl.when(i == 0)
    def _init():
        load_ref[...] = tpe_blk
        spe_acc[...] = spe_blk

    @pl.when(i != 0)
    def _acc():
        load_ref[...] += tpe_blk
        spe_acc[...] += spe_blk

    @pl.when(i == nblocks - 1)
    def _fin():
        n = jnp.float32(_N)
        tpe = load_ref[...] / n
        spe = spe_acc[...] / n
        bal_ref[...] = jnp.sum(tpe * spe).reshape(1, 1) * (_BLW * _E)


@functools.partial(jax.jit, static_argnames=("interpret",))
def kernel(x, W1, b1, W2, b2, keys, interpret=False):
    n, d_in = x.shape
    d_hid = W1.shape[0]
    d_out = W2.shape[0]
    e = keys.shape[0]
    blk = _BLK
    grid = n // blk

    b1r = b1[None, :]
    b2r = b2[None, :]

    out_shapes = (
        jax.ShapeDtypeStruct((n, _K), jnp.int32),
        jax.ShapeDtypeStruct((n, _K), jnp.float32),
        jax.ShapeDtypeStruct((n, _K, d_out), jnp.float32),
        jax.ShapeDtypeStruct((1, e), jnp.float32),
        jax.ShapeDtypeStruct((1, 1), jnp.float32),
    )
    in_specs = [
        pl.BlockSpec((blk, d_in), lambda i: (i, 0)),
        pl.BlockSpec((d_hid, d_in), lambda i: (0, 0)),
        pl.BlockSpec((1, d_hid), lambda i: (0, 0)),
        pl.BlockSpec((d_out, d_hid), lambda i: (0, 0)),
        pl.BlockSpec((1, d_out), lambda i: (0, 0)),
        pl.BlockSpec((e, d_out), lambda i: (0, 0)),
    ]
    out_specs = (
        pl.BlockSpec((blk, _K), lambda i: (i, 0)),
        pl.BlockSpec((blk, _K), lambda i: (i, 0)),
        pl.BlockSpec((blk, _K, d_out), lambda i: (i, 0, 0)),
        pl.BlockSpec((1, e), lambda i: (0, 0)),
        pl.BlockSpec((1, 1), lambda i: (0, 0)),
    )

    idx, scr, sel, load2d, bal = pl.pallas_call(
        _router_kernel,
        grid=(grid,),
        in_specs=in_specs,
        out_specs=out_specs,
        out_shape=out_shapes,
        scratch_shapes=[pltpu.VMEM((1, e), jnp.float32)],
        interpret=interpret,
    )(x, W1, b1r, W2, b2r, keys)

    top_k_indices = idx
    top_k_scores = scr
    selected_keys = sel
    load = load2d.reshape(e)
    balance_loss = bal.reshape(())
    importance = jnp.float32(0.0)
    return (top_k_indices, top_k_scores, balance_loss, load, importance,
            selected_keys)


# speedup vs baseline: 3.2278x; 1.0475x over previous
import functools

import jax
import jax.numpy as jnp
from jax.experimental import pallas as pl
from jax.experimental.pallas import tpu as pltpu

_N = 8192
_E = 16
_K = 2
_BLW = 0.01
_BLK = 1024


def _dot_t(a, b):
    return jax.lax.dot_general(
        a, b, (((1,), (1,)), ((), ())),
        precision=jax.lax.Precision.DEFAULT,
        preferred_element_type=jnp.float32)


def _router_kernel(x_ref, w1_ref, b1_ref, w2_ref, b2_ref,
                   keys_ref, idx_ref, scr_ref, sel_ref, load_ref, bal_ref,
                   spe_acc):
    i = pl.program_id(0)
    nblocks = pl.num_programs(0)
    blk = x_ref.shape[0]
    e = keys_ref.shape[0]
    d_out = keys_ref.shape[1]

    hp = jax.lax.Precision.DEFAULT
    h1 = jnp.maximum(_dot_t(x_ref[...], w1_ref[...]) + b1_ref[...], 0.0)
    h = _dot_t(h1, w2_ref[...]) + b2_ref[...]
    logits = _dot_t(h, keys_ref[...])

    m = jnp.max(logits, axis=-1, keepdims=True)
    ex = jnp.exp(logits - m)
    denom = jnp.sum(ex, axis=-1, keepdims=True)
    p = ex / denom

    cols = jax.lax.broadcasted_iota(jnp.int32, (blk, e), 1)
    i1 = jnp.min(jnp.where(logits == m, cols, e), axis=-1, keepdims=True)
    masked = jnp.where(cols == i1, -jnp.inf, logits)
    m2 = jnp.max(masked, axis=-1, keepdims=True)
    i2 = jnp.min(jnp.where(masked == m2, cols, e), axis=-1, keepdims=True)

    oh1 = (cols == i1).astype(jnp.float32)
    oh2 = (cols == i2).astype(jnp.float32)
    p1 = jnp.sum(p * oh1, axis=-1, keepdims=True)
    p2 = jnp.sum(p * oh2, axis=-1, keepdims=True)
    tot = p1 + p2

    idx_ref[...] = jnp.concatenate([i1, i2], axis=1)
    scr_ref[...] = jnp.concatenate([p1 / tot, p2 / tot], axis=1)

    sel_ref[:, 0, :] = jnp.dot(oh1, keys_ref[...], precision=hp)
    sel_ref[:, 1, :] = jnp.dot(oh2, keys_ref[...], precision=hp)

    tpe_blk = jnp.sum(oh1 + oh2, axis=0, keepdims=True)
    spe_blk = jnp.sum(p, axis=0, keepdims=True)

    @pl.when(i == 0)
    def _init():
        load_ref[...] = tpe_blk
        spe_acc[...] = spe_blk

    @pl.when(i != 0)
    def _acc():
        load_ref[...] += tpe_blk
        spe_acc[...] += spe_blk

    @pl.when(i == nblocks - 1)
    def _fin():
        n = jnp.float32(_N)
        tpe = load_ref[...] / n
        spe = spe_acc[...] / n
        bal_ref[...] = jnp.sum(tpe * spe).reshape(1, 1) * (_BLW * _E)


@functools.partial(jax.jit, static_argnames=("interpret",))
def kernel(x, W1, b1, W2, b2, keys, interpret=False):
    n, d_in = x.shape
    d_hid = W1.shape[0]
    d_out = W2.shape[0]
    e = keys.shape[0]
    blk = _BLK
    grid = n // blk

    b1r = b1[None, :]
    b2r = b2[None, :]

    out_shapes = (
        jax.ShapeDtypeStruct((n, _K), jnp.int32),
        jax.ShapeDtypeStruct((n, _K), jnp.float32),
        jax.ShapeDtypeStruct((n, _K, d_out), jnp.float32),
        jax.ShapeDtypeStruct((1, e), jnp.float32),
        jax.ShapeDtypeStruct((1, 1), jnp.float32),
    )
    in_specs = [
        pl.BlockSpec((blk, d_in), lambda i: (i, 0)),
        pl.BlockSpec((d_hid, d_in), lambda i: (0, 0)),
        pl.BlockSpec((1, d_hid), lambda i: (0, 0)),
        pl.BlockSpec((d_out, d_hid), lambda i: (0, 0)),
        pl.BlockSpec((1, d_out), lambda i: (0, 0)),
        pl.BlockSpec((e, d_out), lambda i: (0, 0)),
    ]
    out_specs = (
        pl.BlockSpec((blk, _K), lambda i: (i, 0)),
        pl.BlockSpec((blk, _K), lambda i: (i, 0)),
        pl.BlockSpec((blk, _K, d_out), lambda i: (i, 0, 0)),
        pl.BlockSpec((1, e), lambda i: (0, 0)),
        pl.BlockSpec((1, 1), lambda i: (0, 0)),
    )

    idx, scr, sel, load2d, bal = pl.pallas_call(
        _router_kernel,
        grid=(grid,),
        in_specs=in_specs,
        out_specs=out_specs,
        out_shape=out_shapes,
        scratch_shapes=[pltpu.VMEM((1, e), jnp.float32)],
        interpret=interpret,
    )(x, W1, b1r, W2, b2r, keys)

    top_k_indices = idx
    top_k_scores = scr
    selected_keys = sel
    load = load2d.reshape(e)
    balance_loss = bal.reshape(())
    importance = jnp.float32(0.0)
    return (top_k_indices, top_k_scores, balance_loss, load, importance,
            selected_keys)
